# baseline (speedup 1.0000x reference)
import functools

import jax
import jax.numpy as jnp
from jax import lax
from jax.experimental import pallas as pl
from jax.experimental.pallas import tpu as pltpu

_EPS = 1e-5
_PAD = 128


def _conv_stats_kernel(x_ref, a_ref, conv_ref, st_ref, *, H, W, taps):
    P = H * W
    Cin = x_ref.shape[1]
    Cout = conv_ref.shape[1]

    xb = x_ref[0].astype(jnp.bfloat16)
    xp = jnp.pad(xb, ((0, 0), (_PAD, _PAD)))

    w_idx = lax.broadcasted_iota(jnp.int32, (Cin, P), 1) % W
    mask_l = (w_idx > 0).astype(jnp.bfloat16)
    mask_r = (w_idx < W - 1).astype(jnp.bfloat16)

    acc = jnp.zeros((Cout, P), jnp.float32)
    for t, (dh, dw) in enumerate(taps):
        s = dh * W + dw
        p = lax.slice(xp, (0, _PAD + s), (Cin, _PAD + s + P))
        if dw == 1:
            p = p * mask_r
        elif dw == -1:
            p = p * mask_l
        acc = acc + jnp.dot(a_ref[t], p, preferred_element_type=jnp.float32)
    conv_ref[0] = acc.astype(jnp.bfloat16)

    ssum = jnp.sum(acc, axis=1, keepdims=True)
    ssq = jnp.sum(acc * acc, axis=1, keepdims=True)
    st = jnp.concatenate(
        [jnp.broadcast_to(ssum, (1, Cout, 128)),
         jnp.broadcast_to(ssq, (1, Cout, 128))], axis=0)

    @pl.when(pl.program_id(0) == 0)
    def _init():
        st_ref[...] = st

    @pl.when(pl.program_id(0) > 0)
    def _accum():
        st_ref[...] += st


def _bn_relu_kernel(conv_ref, st_ref, gb_ref, o_ref, *, count):
    inv_n = 1.0 / count
    mean = st_ref[0, :, 0:1] * inv_n
    var = jnp.maximum(st_ref[1, :, 0:1] * inv_n - mean * mean, 0.0)
    inv_std = lax.rsqrt(var + _EPS)
    scale = gb_ref[0, :, 0:1] * inv_std
    shift = gb_ref[1, :, 0:1] - mean * scale
    y = conv_ref[0].astype(jnp.float32) * scale + shift
    o_ref[0] = jnp.maximum(y, 0.0)


@jax.jit
def _conv_bn_relu(x_nchw, weight_oihw, gamma, beta):
    N, Cin, H, W = x_nchw.shape
    Cout, _, KH, KW = weight_oihw.shape
    P = H * W
    taps = tuple((kh - (KH - 1) // 2, kw - (KW - 1) // 2)
                 for kh in range(KH) for kw in range(KW))

    xf = x_nchw.reshape(N, Cin, P)
    a_mat = jnp.transpose(weight_oihw, (2, 3, 0, 1)).reshape(KH * KW, Cout, Cin)
    a_mat = a_mat.astype(jnp.bfloat16)
    gb = jnp.broadcast_to(
        jnp.stack([gamma.astype(jnp.float32), beta.astype(jnp.float32)])[:, :, None],
        (2, Cout, 128))

    cparams = pltpu.CompilerParams(
        dimension_semantics=("arbitrary",),
        vmem_limit_bytes=48 * 1024 * 1024,
    )

    conv, stats = pl.pallas_call(
        functools.partial(_conv_stats_kernel, H=H, W=W, taps=taps),
        grid=(N,),
        out_shape=(
            jax.ShapeDtypeStruct((N, Cout, P), jnp.bfloat16),
            jax.ShapeDtypeStruct((2, Cout, 128), jnp.float32),
        ),
        in_specs=[
            pl.BlockSpec((1, Cin, P), lambda n: (n, 0, 0)),
            pl.BlockSpec((KH * KW, Cout, Cin), lambda n: (0, 0, 0)),
        ],
        out_specs=(
            pl.BlockSpec((1, Cout, P), lambda n: (n, 0, 0)),
            pl.BlockSpec((2, Cout, 128), lambda n: (0, 0, 0)),
        ),
        compiler_params=cparams,
    )(xf, a_mat)

    out = pl.pallas_call(
        functools.partial(_bn_relu_kernel, count=N * P),
        grid=(N,),
        out_shape=jax.ShapeDtypeStruct((N, Cout, P), jnp.float32),
        in_specs=[
            pl.BlockSpec((1, Cout, P), lambda n: (n, 0, 0)),
            pl.BlockSpec((2, Cout, 128), lambda n: (0, 0, 0)),
            pl.BlockSpec((2, Cout, 128), lambda n: (0, 0, 0)),
        ],
        out_specs=pl.BlockSpec((1, Cout, P), lambda n: (n, 0, 0)),
        compiler_params=cparams,
    )(conv, stats, gb)

    return out.reshape(N, Cout, H, W)


def kernel(x_nchw, weight_oihw, bias, gamma, beta):
    del bias
    return _conv_bn_relu(x_nchw, weight_oihw, gamma, beta)


# speedup vs baseline: 1.6928x; 1.6928x over previous
import functools

import jax
import jax.numpy as jnp
from jax import lax
from jax.experimental import pallas as pl
from jax.experimental.pallas import tpu as pltpu

_EPS = 1e-5
_PAD = 128


def _conv_stats_kernel(x_ref, a_ref, conv_ref, st_ref, *, H, W, taps):
    P = H * W
    Cin = x_ref.shape[1]
    Cout = conv_ref.shape[1]

    xb = x_ref[0].astype(jnp.bfloat16).reshape(Cin, P)
    xp = jnp.pad(xb, ((0, 0), (_PAD, _PAD)))

    w_idx = lax.broadcasted_iota(jnp.int32, (Cin, P), 1) % W
    mask_l = (w_idx > 0).astype(jnp.bfloat16)
    mask_r = (w_idx < W - 1).astype(jnp.bfloat16)

    acc = jnp.zeros((Cout, P), jnp.float32)
    for t, (dh, dw) in enumerate(taps):
        s = dh * W + dw
        p = lax.slice(xp, (0, _PAD + s), (Cin, _PAD + s + P))
        if dw == 1:
            p = p * mask_r
        elif dw == -1:
            p = p * mask_l
        acc = acc + jnp.dot(a_ref[t], p, preferred_element_type=jnp.float32)
    conv_ref[0] = acc.astype(jnp.bfloat16)

    ssum = jnp.sum(acc, axis=1, keepdims=True)
    ssq = jnp.sum(acc * acc, axis=1, keepdims=True)
    st = jnp.concatenate(
        [jnp.broadcast_to(ssum, (1, Cout, 128)),
         jnp.broadcast_to(ssq, (1, Cout, 128))], axis=0)

    @pl.when(pl.program_id(0) == 0)
    def _init():
        st_ref[...] = st

    @pl.when(pl.program_id(0) > 0)
    def _accum():
        st_ref[...] += st


def _bn_relu_kernel(conv_ref, st_ref, gb_ref, o_ref, *, count):
    Cout, H, W = o_ref.shape[1:]
    inv_n = 1.0 / count
    mean = st_ref[0, :, 0:1] * inv_n
    var = jnp.maximum(st_ref[1, :, 0:1] * inv_n - mean * mean, 0.0)
    inv_std = lax.rsqrt(var + _EPS)
    scale = gb_ref[0, :, 0:1] * inv_std
    shift = gb_ref[1, :, 0:1] - mean * scale
    y = conv_ref[0].astype(jnp.float32) * scale + shift
    o_ref[0] = jnp.maximum(y, 0.0).reshape(Cout, H, W)


@jax.jit
def _conv_bn_relu(x_nchw, weight_oihw, gamma, beta):
    N, Cin, H, W = x_nchw.shape
    Cout, _, KH, KW = weight_oihw.shape
    P = H * W
    taps = tuple((kh - (KH - 1) // 2, kw - (KW - 1) // 2)
                 for kh in range(KH) for kw in range(KW))

    a_mat = jnp.transpose(weight_oihw, (2, 3, 0, 1)).reshape(KH * KW, Cout, Cin)
    a_mat = a_mat.astype(jnp.bfloat16)
    gb = jnp.broadcast_to(
        jnp.stack([gamma.astype(jnp.float32), beta.astype(jnp.float32)])[:, :, None],
        (2, Cout, 128))

    cparams = pltpu.CompilerParams(
        dimension_semantics=("arbitrary",),
        vmem_limit_bytes=48 * 1024 * 1024,
    )

    conv, stats = pl.pallas_call(
        functools.partial(_conv_stats_kernel, H=H, W=W, taps=taps),
        grid=(N,),
        out_shape=(
            jax.ShapeDtypeStruct((N, Cout, P), jnp.bfloat16),
            jax.ShapeDtypeStruct((2, Cout, 128), jnp.float32),
        ),
        in_specs=[
            pl.BlockSpec((1, Cin, H, W), lambda n: (n, 0, 0, 0)),
            pl.BlockSpec((KH * KW, Cout, Cin), lambda n: (0, 0, 0)),
        ],
        out_specs=(
            pl.BlockSpec((1, Cout, P), lambda n: (n, 0, 0)),
            pl.BlockSpec((2, Cout, 128), lambda n: (0, 0, 0)),
        ),
        compiler_params=cparams,
    )(x_nchw, a_mat)

    out = pl.pallas_call(
        functools.partial(_bn_relu_kernel, count=N * P),
        grid=(N,),
        out_shape=jax.ShapeDtypeStruct((N, Cout, H, W), jnp.float32),
        in_specs=[
            pl.BlockSpec((1, Cout, P), lambda n: (n, 0, 0)),
            pl.BlockSpec((2, Cout, 128), lambda n: (0, 0, 0)),
            pl.BlockSpec((2, Cout, 128), lambda n: (0, 0, 0)),
        ],
        out_specs=pl.BlockSpec((1, Cout, H, W), lambda n: (n, 0, 0, 0)),
        compiler_params=cparams,
    )(conv, stats, gb)

    return out


def kernel(x_nchw, weight_oihw, bias, gamma, beta):
    del bias
    return _conv_bn_relu(x_nchw, weight_oihw, gamma, beta)


# speedup vs baseline: 1.8135x; 1.0713x over previous
import functools

import jax
import jax.numpy as jnp
from jax import lax
from jax.experimental import pallas as pl
from jax.experimental.pallas import tpu as pltpu

_EPS = 1e-5
_PAD = 128


def _conv_stats_kernel(x_ref, a_ref, conv_ref, st_ref, *, H, W, dws):
    P = H * W
    W2 = 128
    P2 = H * W2
    Cin = x_ref.shape[1]
    Cout = conv_ref.shape[1]

    xb3 = x_ref[0].astype(jnp.bfloat16)
    xw = jnp.pad(xb3, ((0, 0), (0, 0), (0, W2 - W)))
    xs = xw.reshape(Cin, P2)
    xsp = jnp.pad(xs, ((0, 0), (2 * W2, 2 * W2)))

    acc = jnp.zeros((Cout, P2), jnp.float32)
    for g, dw in enumerate(dws):
        ys = lax.slice(xsp, (0, W2 + dw), (Cin, W2 + dw + P2 + 2 * W2))
        b = jnp.concatenate(
            [lax.slice(ys, (0, (dh + 1) * W2), (Cin, (dh + 1) * W2 + P2))
             for dh in (-1, 0, 1)], axis=0)
        acc = acc + jnp.dot(a_ref[g], b, preferred_element_type=jnp.float32)

    cc = jnp.concatenate(
        [lax.slice(acc, (0, h * W2), (Cout, h * W2 + W)) for h in range(H)],
        axis=1)
    conv_ref[0] = cc.astype(jnp.bfloat16)

    ssum = jnp.sum(cc, axis=1, keepdims=True)
    ssq = jnp.sum(cc * cc, axis=1, keepdims=True)
    st = jnp.concatenate(
        [jnp.broadcast_to(ssum, (1, Cout, 128)),
         jnp.broadcast_to(ssq, (1, Cout, 128))], axis=0)

    @pl.when(pl.program_id(0) == 0)
    def _init():
        st_ref[...] = st

    @pl.when(pl.program_id(0) > 0)
    def _accum():
        st_ref[...] += st


def _bn_relu_kernel(conv_ref, st_ref, gb_ref, o_ref, *, count):
    Cout, H, W = o_ref.shape[1:]
    inv_n = 1.0 / count
    mean = st_ref[0, :, 0:1] * inv_n
    var = jnp.maximum(st_ref[1, :, 0:1] * inv_n - mean * mean, 0.0)
    inv_std = lax.rsqrt(var + _EPS)
    scale = gb_ref[0, :, 0:1] * inv_std
    shift = gb_ref[1, :, 0:1] - mean * scale
    y = conv_ref[0].astype(jnp.float32) * scale + shift
    o_ref[0] = jnp.maximum(y, 0.0).reshape(Cout, H, W)


@jax.jit
def _conv_bn_relu(x_nchw, weight_oihw, gamma, beta):
    N, Cin, H, W = x_nchw.shape
    Cout, _, KH, KW = weight_oihw.shape
    P = H * W
    dws = tuple(kw - (KW - 1) // 2 for kw in range(KW))

    a_mat = jnp.transpose(weight_oihw, (3, 0, 2, 1)).reshape(KW, Cout, KH * Cin)
    a_mat = a_mat.astype(jnp.bfloat16)
    gb = jnp.broadcast_to(
        jnp.stack([gamma.astype(jnp.float32), beta.astype(jnp.float32)])[:, :, None],
        (2, Cout, 128))

    cparams = pltpu.CompilerParams(
        dimension_semantics=("arbitrary",),
        vmem_limit_bytes=48 * 1024 * 1024,
    )

    conv, stats = pl.pallas_call(
        functools.partial(_conv_stats_kernel, H=H, W=W, dws=dws),
        grid=(N,),
        out_shape=(
            jax.ShapeDtypeStruct((N, Cout, P), jnp.bfloat16),
            jax.ShapeDtypeStruct((2, Cout, 128), jnp.float32),
        ),
        in_specs=[
            pl.BlockSpec((1, Cin, H, W), lambda n: (n, 0, 0, 0)),
            pl.BlockSpec((KW, Cout, KH * Cin), lambda n: (0, 0, 0)),
        ],
        out_specs=(
            pl.BlockSpec((1, Cout, P), lambda n: (n, 0, 0)),
            pl.BlockSpec((2, Cout, 128), lambda n: (0, 0, 0)),
        ),
        compiler_params=cparams,
    )(x_nchw, a_mat)

    out = pl.pallas_call(
        functools.partial(_bn_relu_kernel, count=N * P),
        grid=(N,),
        out_shape=jax.ShapeDtypeStruct((N, Cout, H, W), jnp.float32),
        in_specs=[
            pl.BlockSpec((1, Cout, P), lambda n: (n, 0, 0)),
            pl.BlockSpec((2, Cout, 128), lambda n: (0, 0, 0)),
            pl.BlockSpec((2, Cout, 128), lambda n: (0, 0, 0)),
        ],
        out_specs=pl.BlockSpec((1, Cout, H, W), lambda n: (n, 0, 0, 0)),
        compiler_params=cparams,
    )(conv, stats, gb)

    return out


def kernel(x_nchw, weight_oihw, bias, gamma, beta):
    del bias
    return _conv_bn_relu(x_nchw, weight_oihw, gamma, beta)


# speedup vs baseline: 2.6927x; 1.4848x over previous
import functools

import jax
import jax.numpy as jnp
from jax import lax
from jax.experimental import pallas as pl
from jax.experimental.pallas import tpu as pltpu

_EPS = 1e-5


def _conv_stats_kernel(x_ref, a_ref, conv_ref, st_ref, *, H, W, dws):
    W2 = 2 * W
    P2 = H * W2
    Cin = x_ref.shape[1]
    Cout = conv_ref.shape[1]

    xb = x_ref[...].astype(jnp.bfloat16)
    xw = jnp.concatenate([xb[0], xb[1]], axis=2)
    xs = xw.reshape(Cin, P2)
    xsp = jnp.pad(xs, ((0, 0), (2 * W2, 2 * W2)))

    w_idx = lax.broadcasted_iota(jnp.int32, (Cin, P2 + 2 * W2), 1) % W
    mask_l = (w_idx > 0).astype(jnp.bfloat16)
    mask_r = (w_idx < W - 1).astype(jnp.bfloat16)

    acc = jnp.zeros((Cout, P2), jnp.float32)
    for g, dw in enumerate(dws):
        ys = lax.slice(xsp, (0, W2 + dw), (Cin, W2 + dw + P2 + 2 * W2))
        if dw == 1:
            ys = ys * mask_r
        elif dw == -1:
            ys = ys * mask_l
        b = jnp.concatenate(
            [lax.slice(ys, (0, (dh + 1) * W2), (Cin, (dh + 1) * W2 + P2))
             for dh in (-1, 0, 1)], axis=0)
        acc = acc + jnp.dot(a_ref[g], b, preferred_element_type=jnp.float32)

    conv_ref[0] = acc.astype(jnp.bfloat16).reshape(Cout, H, W2)

    ssum = jnp.sum(acc, axis=1, keepdims=True)
    ssq = jnp.sum(acc * acc, axis=1, keepdims=True)
    st = jnp.concatenate(
        [jnp.broadcast_to(ssum, (1, Cout, 128)),
         jnp.broadcast_to(ssq, (1, Cout, 128))], axis=0)

    @pl.when(pl.program_id(0) == 0)
    def _init():
        st_ref[...] = st

    @pl.when(pl.program_id(0) > 0)
    def _accum():
        st_ref[...] += st


def _bn_relu_kernel(conv_ref, st_ref, gb_ref, o_ref, *, count):
    Cout, H, W = o_ref.shape[1:]
    inv_n = 1.0 / count
    mean = st_ref[0, :, 0:1] * inv_n
    var = jnp.maximum(st_ref[1, :, 0:1] * inv_n - mean * mean, 0.0)
    inv_std = lax.rsqrt(var + _EPS)
    scale = (gb_ref[0, :, 0:1] * inv_std)[:, :, None]
    shift = (gb_ref[1, :, 0:1] - mean * gb_ref[0, :, 0:1] * inv_std)[:, :, None]
    y = jnp.maximum(conv_ref[0].astype(jnp.float32) * scale + shift, 0.0)
    o_ref[0] = lax.slice(y, (0, 0, 0), (Cout, H, W))
    o_ref[1] = lax.slice(y, (0, 0, W), (Cout, H, 2 * W))


@jax.jit
def _conv_bn_relu(x_nchw, weight_oihw, gamma, beta):
    N, Cin, H, W = x_nchw.shape
    Cout, _, KH, KW = weight_oihw.shape
    N2 = N // 2
    dws = tuple(kw - (KW - 1) // 2 for kw in range(KW))

    a_mat = jnp.transpose(weight_oihw, (3, 0, 2, 1)).reshape(KW, Cout, KH * Cin)
    a_mat = a_mat.astype(jnp.bfloat16)
    gb = jnp.broadcast_to(
        jnp.stack([gamma.astype(jnp.float32), beta.astype(jnp.float32)])[:, :, None],
        (2, Cout, 128))

    cparams = pltpu.CompilerParams(
        dimension_semantics=("arbitrary",),
        vmem_limit_bytes=48 * 1024 * 1024,
    )

    conv, stats = pl.pallas_call(
        functools.partial(_conv_stats_kernel, H=H, W=W, dws=dws),
        grid=(N2,),
        out_shape=(
            jax.ShapeDtypeStruct((N2, Cout, H, 2 * W), jnp.bfloat16),
            jax.ShapeDtypeStruct((2, Cout, 128), jnp.float32),
        ),
        in_specs=[
            pl.BlockSpec((2, Cin, H, W), lambda n: (n, 0, 0, 0)),
            pl.BlockSpec((KW, Cout, KH * Cin), lambda n: (0, 0, 0)),
        ],
        out_specs=(
            pl.BlockSpec((1, Cout, H, 2 * W), lambda n: (n, 0, 0, 0)),
            pl.BlockSpec((2, Cout, 128), lambda n: (0, 0, 0)),
        ),
        compiler_params=cparams,
    )(x_nchw, a_mat)

    out = pl.pallas_call(
        functools.partial(_bn_relu_kernel, count=N * H * W),
        grid=(N2,),
        out_shape=jax.ShapeDtypeStruct((N, Cout, H, W), jnp.float32),
        in_specs=[
            pl.BlockSpec((1, Cout, H, 2 * W), lambda n: (n, 0, 0, 0)),
            pl.BlockSpec((2, Cout, 128), lambda n: (0, 0, 0)),
            pl.BlockSpec((2, Cout, 128), lambda n: (0, 0, 0)),
        ],
        out_specs=pl.BlockSpec((2, Cout, H, W), lambda n: (n, 0, 0, 0)),
        compiler_params=cparams,
    )(conv, stats, gb)

    return out


def kernel(x_nchw, weight_oihw, bias, gamma, beta):
    del bias
    return _conv_bn_relu(x_nchw, weight_oihw, gamma, beta)


# speedup vs baseline: 3.0518x; 1.1334x over previous
import functools

import jax
import jax.numpy as jnp
from jax import lax
from jax.experimental import pallas as pl
from jax.experimental.pallas import tpu as pltpu

_EPS = 1e-5


def _fused_kernel(x_ref, a_ref, gb_ref, o_ref, conv_scr, st_scr,
                  *, H, W, dws, count):
    W2 = 2 * W
    P2 = H * W2
    Cin = x_ref.shape[1]
    Cout = o_ref.shape[1]
    ph = pl.program_id(0)
    n = pl.program_id(1)

    @pl.when(ph == 0)
    def _conv_phase():
        xb = x_ref[...].astype(jnp.bfloat16)
        xw = jnp.concatenate([xb[0], xb[1]], axis=2)
        xs = xw.reshape(Cin, P2)
        xsp = jnp.pad(xs, ((0, 0), (2 * W2, 2 * W2)))

        w_idx = lax.broadcasted_iota(jnp.int32, (Cin, P2 + 2 * W2), 1) % W
        mask_l = (w_idx > 0).astype(jnp.bfloat16)
        mask_r = (w_idx < W - 1).astype(jnp.bfloat16)

        acc = jnp.zeros((Cout, P2), jnp.float32)
        for g, dw in enumerate(dws):
            ys = lax.slice(xsp, (0, W2 + dw), (Cin, W2 + dw + P2 + 2 * W2))
            if dw == 1:
                ys = ys * mask_r
            elif dw == -1:
                ys = ys * mask_l
            b = jnp.concatenate(
                [lax.slice(ys, (0, (dh + 1) * W2), (Cin, (dh + 1) * W2 + P2))
                 for dh in (-1, 0, 1)], axis=0)
            acc = acc + jnp.dot(a_ref[g], b, preferred_element_type=jnp.float32)

        conv_scr[n] = acc.astype(jnp.bfloat16).reshape(Cout, H, W2)

        ssum = jnp.sum(acc, axis=1, keepdims=True)
        ssq = jnp.sum(acc * acc, axis=1, keepdims=True)
        st = jnp.concatenate(
            [jnp.broadcast_to(ssum, (1, Cout, 128)),
             jnp.broadcast_to(ssq, (1, Cout, 128))], axis=0)

        @pl.when(n == 0)
        def _init():
            st_scr[...] = st

        @pl.when(n > 0)
        def _accum():
            st_scr[...] += st

    @pl.when(ph == 1)
    def _bn_phase():
        inv_n = 1.0 / count
        mean = st_scr[0, :, 0:1] * inv_n
        var = jnp.maximum(st_scr[1, :, 0:1] * inv_n - mean * mean, 0.0)
        inv_std = lax.rsqrt(var + _EPS)
        scale = (gb_ref[0, :, 0:1] * inv_std)[:, :, None]
        shift = (gb_ref[1, :, 0:1] - mean * gb_ref[0, :, 0:1] * inv_std)[:, :, None]
        y = jnp.maximum(conv_scr[n].astype(jnp.float32) * scale + shift, 0.0)
        o_ref[0] = lax.slice(y, (0, 0, 0), (Cout, H, W))
        o_ref[1] = lax.slice(y, (0, 0, W), (Cout, H, W2))


@jax.jit
def _conv_bn_relu(x_nchw, weight_oihw, gamma, beta):
    N, Cin, H, W = x_nchw.shape
    Cout, _, KH, KW = weight_oihw.shape
    N2 = N // 2
    dws = tuple(kw - (KW - 1) // 2 for kw in range(KW))

    a_mat = jnp.transpose(weight_oihw, (3, 0, 2, 1)).reshape(KW, Cout, KH * Cin)
    a_mat = a_mat.astype(jnp.bfloat16)
    gb = jnp.broadcast_to(
        jnp.stack([gamma.astype(jnp.float32), beta.astype(jnp.float32)])[:, :, None],
        (2, Cout, 128))

    out = pl.pallas_call(
        functools.partial(_fused_kernel, H=H, W=W, dws=dws, count=N * H * W),
        grid=(2, N2),
        out_shape=jax.ShapeDtypeStruct((N, Cout, H, W), jnp.float32),
        in_specs=[
            pl.BlockSpec((2, Cin, H, W), lambda ph, n: (n * (1 - ph), 0, 0, 0)),
            pl.BlockSpec((KW, Cout, KH * Cin), lambda ph, n: (0, 0, 0)),
            pl.BlockSpec((2, Cout, 128), lambda ph, n: (0, 0, 0)),
        ],
        out_specs=pl.BlockSpec((2, Cout, H, W), lambda ph, n: (n * ph, 0, 0, 0)),
        scratch_shapes=[
            pltpu.VMEM((N2, Cout, H, 2 * W), jnp.bfloat16),
            pltpu.VMEM((2, Cout, 128), jnp.float32),
        ],
        compiler_params=pltpu.CompilerParams(
            dimension_semantics=("arbitrary", "arbitrary"),
            vmem_limit_bytes=48 * 1024 * 1024,
        ),
    )(x_nchw, a_mat, gb)

    return out


def kernel(x_nchw, weight_oihw, bias, gamma, beta):
    del bias
    return _conv_bn_relu(x_nchw, weight_oihw, gamma, beta)


# speedup vs baseline: 3.0908x; 1.0128x over previous
import functools

import jax
import jax.numpy as jnp
from jax import lax
from jax.experimental import pallas as pl
from jax.experimental.pallas import tpu as pltpu

_EPS = 1e-5


def _fused_kernel(x_ref, a_ref, gb_ref, o_ref, conv_scr, st_scr,
                  *, H, W, dws, count):
    W2 = 2 * W
    P2 = H * W2
    Cin = x_ref.shape[1]
    Cout = o_ref.shape[1]
    ph = pl.program_id(0)
    n = pl.program_id(1)

    @pl.when(ph == 0)
    def _conv_phase():
        w_idx = lax.broadcasted_iota(jnp.int32, (Cin, P2 + 2 * W2), 1) % W
        mask_l = (w_idx > 0).astype(jnp.bfloat16)
        mask_r = (w_idx < W - 1).astype(jnp.bfloat16)

        for j in range(x_ref.shape[0] // 2):
            xb = x_ref[pl.ds(2 * j, 2)].astype(jnp.bfloat16)
            xw = jnp.concatenate([xb[0], xb[1]], axis=2)
            xs = xw.reshape(Cin, P2)
            xsp = jnp.pad(xs, ((0, 0), (2 * W2, 2 * W2)))

            acc = jnp.zeros((Cout, P2), jnp.float32)
            for g, dw in enumerate(dws):
                ys = lax.slice(xsp, (0, W2 + dw), (Cin, W2 + dw + P2 + 2 * W2))
                if dw == 1:
                    ys = ys * mask_r
                elif dw == -1:
                    ys = ys * mask_l
                b = jnp.concatenate(
                    [lax.slice(ys, (0, (dh + 1) * W2), (Cin, (dh + 1) * W2 + P2))
                     for dh in (-1, 0, 1)], axis=0)
                acc = acc + jnp.dot(a_ref[g], b,
                                    preferred_element_type=jnp.float32)

            conv_scr[2 * n + j] = acc.astype(jnp.bfloat16).reshape(Cout, H, W2)

            ssum = jnp.sum(acc, axis=1, keepdims=True)
            ssq = jnp.sum(acc * acc, axis=1, keepdims=True)
            st = jnp.concatenate(
                [jnp.broadcast_to(ssum, (1, Cout, 128)),
                 jnp.broadcast_to(ssq, (1, Cout, 128))], axis=0)

            if j == 0:
                @pl.when(n == 0)
                def _init():
                    st_scr[...] = st

                @pl.when(n > 0)
                def _accum():
                    st_scr[...] += st
            else:
                st_scr[...] += st

    @pl.when(ph == 1)
    def _bn_phase():
        inv_n = 1.0 / count
        mean = st_scr[0, :, 0:1] * inv_n
        var = jnp.maximum(st_scr[1, :, 0:1] * inv_n - mean * mean, 0.0)
        inv_std = lax.rsqrt(var + _EPS)
        scale = (gb_ref[0, :, 0:1] * inv_std)[:, :, None]
        shift = (gb_ref[1, :, 0:1] - mean * gb_ref[0, :, 0:1] * inv_std)[:, :, None]
        for j in range(o_ref.shape[0] // 2):
            y = jnp.maximum(
                conv_scr[2 * n + j].astype(jnp.float32) * scale + shift, 0.0)
            o_ref[2 * j] = lax.slice(y, (0, 0, 0), (Cout, H, W))
            o_ref[2 * j + 1] = lax.slice(y, (0, 0, W), (Cout, H, W2))


@jax.jit
def _conv_bn_relu(x_nchw, weight_oihw, gamma, beta):
    N, Cin, H, W = x_nchw.shape
    Cout, _, KH, KW = weight_oihw.shape
    N2 = N // 2
    dws = tuple(kw - (KW - 1) // 2 for kw in range(KW))

    a_mat = jnp.transpose(weight_oihw, (3, 0, 2, 1)).reshape(KW, Cout, KH * Cin)
    a_mat = a_mat.astype(jnp.bfloat16)
    gb = jnp.broadcast_to(
        jnp.stack([gamma.astype(jnp.float32), beta.astype(jnp.float32)])[:, :, None],
        (2, Cout, 128))

    out = pl.pallas_call(
        functools.partial(_fused_kernel, H=H, W=W, dws=dws, count=N * H * W),
        grid=(2, N2 // 2),
        out_shape=jax.ShapeDtypeStruct((N, Cout, H, W), jnp.float32),
        in_specs=[
            pl.BlockSpec((4, Cin, H, W), lambda ph, n: (n * (1 - ph), 0, 0, 0)),
            pl.BlockSpec((KW, Cout, KH * Cin), lambda ph, n: (0, 0, 0)),
            pl.BlockSpec((2, Cout, 128), lambda ph, n: (0, 0, 0)),
        ],
        out_specs=pl.BlockSpec((4, Cout, H, W), lambda ph, n: (n * ph, 0, 0, 0)),
        scratch_shapes=[
            pltpu.VMEM((N2, Cout, H, 2 * W), jnp.bfloat16),
            pltpu.VMEM((2, Cout, 128), jnp.float32),
        ],
        compiler_params=pltpu.CompilerParams(
            dimension_semantics=("arbitrary", "arbitrary"),
            vmem_limit_bytes=56 * 1024 * 1024,
        ),
    )(x_nchw, a_mat, gb)

    return out


def kernel(x_nchw, weight_oihw, bias, gamma, beta):
    del bias
    return _conv_bn_relu(x_nchw, weight_oihw, gamma, beta)
